# baseline (speedup 1.0000x reference)
import jax
import jax.numpy as jnp
from jax import lax
from jax.experimental import pallas as pl
from jax.experimental.pallas import tpu as pltpu
from jax.experimental.pallas import tpu_sc as plsc

_M = 1_000_000
_D = 8
_N = _M * _D
_NW = 32
_PER_W = _N // _NW
_CHUNK = 50_000
_NCHUNK = _PER_W // _CHUNK
_NUPD = 4


def _apply_update(out_hbm, upd_v, idx_vec, wbuf, j):
    off = pl.multiple_of(idx_vec[j] * _D, _D)
    pltpu.sync_copy(out_hbm.at[pl.ds(off, _D)], wbuf.at[pl.ds(0, _D)])
    uv = upd_v[pl.ds(j * 16, 16)]
    wbuf[...] = wbuf[...] + uv
    pltpu.sync_copy(wbuf.at[pl.ds(0, _D)], out_hbm.at[pl.ds(off, _D)])


def _body(x_hbm, upd_hbm, idx_hbm, out_hbm, buf, upd_v, idx_v, wbuf):
    wid = lax.axis_index("s") * 2 + lax.axis_index("c")
    base = wid * _PER_W

    for k in range(_NCHUNK):
        off = base + k * _CHUNK
        pltpu.sync_copy(x_hbm.at[pl.ds(off, _CHUNK)], buf)
        pltpu.sync_copy(buf, out_hbm.at[pl.ds(off, _CHUNK)])

    pltpu.sync_copy(upd_hbm, upd_v)
    pltpu.sync_copy(idx_hbm, idx_v)
    idx_vec = idx_v[...]
    row_lo = base // _D
    row_hi = row_lo + _PER_W // _D
    for j in range(_NUPD):
        idx_j = idx_vec[j]
        own = (idx_j >= row_lo) & (idx_j < row_hi)

        def _rmw(jj=j):
            _apply_update(out_hbm, upd_v, idx_vec, wbuf, jj)

        pl.when(own)(_rmw)


def kernel(x, update, index):
    x_flat = x.reshape(_N)
    upd_pad = jnp.zeros((_NUPD, 16), jnp.float32).at[:, :_D].set(update).reshape(-1)
    idx_pad = jnp.zeros((16,), jnp.int32).at[:_NUPD].set(index)

    mesh = plsc.VectorSubcoreMesh(
        core_axis_name="c", subcore_axis_name="s", num_cores=2, num_subcores=16
    )
    out = pl.kernel(
        _body,
        out_type=jax.ShapeDtypeStruct((_N,), jnp.float32),
        mesh=mesh,
        scratch_types=[
            pltpu.VMEM((_CHUNK,), jnp.float32),
            pltpu.VMEM((_NUPD * 16,), jnp.float32),
            pltpu.VMEM((16,), jnp.int32),
            pltpu.VMEM((16,), jnp.float32),
        ],
    )(x_flat, upd_pad, idx_pad)
    return out.reshape(_M, _D)


# speedup vs baseline: 1.0032x; 1.0032x over previous
import jax
import jax.numpy as jnp
from jax import lax
from jax.experimental import pallas as pl
from jax.experimental.pallas import tpu as pltpu
from jax.experimental.pallas import tpu_sc as plsc

_M = 1_000_000
_D = 8
_N = _M * _D
_NW = 32
_PER_W = _N // _NW
_CHUNK = 10_000
_NCHUNK = _PER_W // _CHUNK
_NBUF = 8
_LAG = _NBUF // 2
_NUPD = 4


def _apply_update(out_hbm, upd_v, idx_vec, wbuf, j):
    off = pl.multiple_of(idx_vec[j] * _D, _D)
    pltpu.sync_copy(out_hbm.at[pl.ds(off, _D)], wbuf.at[pl.ds(0, _D)])
    uv = upd_v[pl.ds(j * 16, 16)]
    wbuf[...] = wbuf[...] + uv
    pltpu.sync_copy(wbuf.at[pl.ds(0, _D)], out_hbm.at[pl.ds(off, _D)])


def _body(x_hbm, upd_hbm, idx_hbm, out_hbm,
          b0, b1, b2, b3, b4, b5, b6, b7, rsems, wsems, upd_v, idx_v, wbuf):
    wid = lax.axis_index("s") * 2 + lax.axis_index("c")
    base = wid * _PER_W
    bufs = (b0, b1, b2, b3, b4, b5, b6, b7)

    def rd(c):
        b = c % _NBUF
        off = base + c * _CHUNK
        return pltpu.make_async_copy(
            x_hbm.at[pl.ds(off, _CHUNK)], bufs[b], rsems.at[b])

    def wr(c):
        b = c % _NBUF
        off = base + c * _CHUNK
        return pltpu.make_async_copy(
            bufs[b], out_hbm.at[pl.ds(off, _CHUNK)], wsems.at[b])

    for c in range(_LAG):
        rd(c).start()
    for c in range(_NCHUNK):
        if c + _LAG < _NCHUNK:
            if c >= _LAG:
                wr(c - _LAG).wait()
            rd(c + _LAG).start()
        rd(c).wait()
        wr(c).start()
    for c in range(_NCHUNK - _LAG, _NCHUNK):
        wr(c).wait()

    pltpu.sync_copy(upd_hbm, upd_v)
    pltpu.sync_copy(idx_hbm, idx_v)
    idx_vec = idx_v[...]
    row_lo = base // _D
    row_hi = row_lo + _PER_W // _D
    for j in range(_NUPD):
        own = (idx_vec[j] >= row_lo) & (idx_vec[j] < row_hi)

        def _rmw(jj=j):
            _apply_update(out_hbm, upd_v, idx_vec, wbuf, jj)

        pl.when(own)(_rmw)


def kernel(x, update, index):
    x_flat = x.reshape(_N)
    upd_pad = jnp.zeros((_NUPD, 16), jnp.float32).at[:, :_D].set(update).reshape(-1)
    idx_pad = jnp.zeros((16,), jnp.int32).at[:_NUPD].set(index)

    mesh = plsc.VectorSubcoreMesh(
        core_axis_name="c", subcore_axis_name="s", num_cores=2, num_subcores=16
    )
    out = pl.kernel(
        _body,
        out_type=jax.ShapeDtypeStruct((_N,), jnp.float32),
        mesh=mesh,
        scratch_types=[
            pltpu.VMEM((_CHUNK,), jnp.float32),
            pltpu.VMEM((_CHUNK,), jnp.float32),
            pltpu.VMEM((_CHUNK,), jnp.float32),
            pltpu.VMEM((_CHUNK,), jnp.float32),
            pltpu.VMEM((_CHUNK,), jnp.float32),
            pltpu.VMEM((_CHUNK,), jnp.float32),
            pltpu.VMEM((_CHUNK,), jnp.float32),
            pltpu.VMEM((_CHUNK,), jnp.float32),
            pltpu.SemaphoreType.DMA((_NBUF,)),
            pltpu.SemaphoreType.DMA((_NBUF,)),
            pltpu.VMEM((_NUPD * 16,), jnp.float32),
            pltpu.VMEM((16,), jnp.int32),
            pltpu.VMEM((16,), jnp.float32),
        ],
    )(x_flat, upd_pad, idx_pad)
    return out.reshape(_M, _D)


# speedup vs baseline: 1.9294x; 1.9233x over previous
import jax
import jax.numpy as jnp
from jax import lax
from jax.experimental import pallas as pl
from jax.experimental.pallas import tpu as pltpu
from jax.experimental.pallas import tpu_sc as plsc

_M = 1_000_000
_D = 8
_RV = 125_000
_CV = 64
_NW = 32
_CH = 200
_NCHUNK = _RV // _CH
_NSLOT = 20
_NBUF = 4
_LAG = _NBUF // 2
_NUPD = 4


def _copy_body(x_hbm, out_hbm, b0, b1, b2, b3, rsems, wsems):
    wid = lax.axis_index("s") * 2 + lax.axis_index("c")
    bufs = (b0, b1, b2, b3)

    def chunk_id(k):
        return wid + k * _NW

    def valid(k):
        return chunk_id(k) < _NCHUNK

    def rd(k):
        b = k % _NBUF
        row0 = chunk_id(k) * _CH
        return pltpu.make_async_copy(
            x_hbm.at[pl.ds(row0, _CH)], bufs[b], rsems.at[b])

    def wr(k):
        b = k % _NBUF
        row0 = chunk_id(k) * _CH
        return pltpu.make_async_copy(
            bufs[b], out_hbm.at[pl.ds(row0, _CH)], wsems.at[b])

    for k in range(_LAG):
        pl.when(valid(k))(lambda k=k: rd(k).start())
    for k in range(_NSLOT):
        if k + _LAG < _NSLOT:
            if k >= _LAG:
                pl.when(valid(k - _LAG))(lambda k=k: wr(k - _LAG).wait())
            pl.when(valid(k + _LAG))(lambda k=k: rd(k + _LAG).start())

        def _proc(k=k):
            rd(k).wait()
            wr(k).start()

        pl.when(valid(k))(_proc)
    for k in range(_NSLOT - _LAG, _NSLOT):
        pl.when(valid(k))(lambda k=k: wr(k).wait())


def _sc_copy(x):
    mesh = plsc.VectorSubcoreMesh(
        core_axis_name="c", subcore_axis_name="s", num_cores=2, num_subcores=16
    )
    out = pl.kernel(
        _copy_body,
        out_type=jax.ShapeDtypeStruct((_RV, _CV), jnp.float32),
        mesh=mesh,
        scratch_types=[
            pltpu.VMEM((_CH, _CV), jnp.float32),
            pltpu.VMEM((_CH, _CV), jnp.float32),
            pltpu.VMEM((_CH, _CV), jnp.float32),
            pltpu.VMEM((_CH, _CV), jnp.float32),
            pltpu.SemaphoreType.DMA((_NBUF,)),
            pltpu.SemaphoreType.DMA((_NBUF,)),
        ],
    )(x.reshape(_RV, _CV))
    return out.reshape(_M, _D)


def _fix_body(idx_ref, table_ref, upd_ref, out_ref, row_vmem, sem):
    def rmw(j, _):
        row = idx_ref[j]
        pltpu.make_async_copy(
            out_ref.at[pl.ds(row, 1)], row_vmem, sem).start()
        pltpu.make_async_copy(
            out_ref.at[pl.ds(row, 1)], row_vmem, sem).wait()
        row_vmem[...] = row_vmem[...] + upd_ref[pl.ds(j, 1), :]
        pltpu.make_async_copy(
            row_vmem, out_ref.at[pl.ds(row, 1)], sem).start()
        pltpu.make_async_copy(
            row_vmem, out_ref.at[pl.ds(row, 1)], sem).wait()
        return ()

    lax.fori_loop(0, _NUPD, rmw, (), unroll=True)


def _tc_fixup(table, update, index):
    grid_spec = pltpu.PrefetchScalarGridSpec(
        num_scalar_prefetch=1,
        grid=(1,),
        in_specs=[
            pl.BlockSpec(memory_space=pl.ANY),
            pl.BlockSpec((_NUPD, _D), lambda i, idx: (0, 0)),
        ],
        out_specs=pl.BlockSpec(memory_space=pl.ANY),
        scratch_shapes=[
            pltpu.VMEM((1, _D), jnp.float32),
            pltpu.SemaphoreType.DMA,
        ],
    )
    return pl.pallas_call(
        _fix_body,
        grid_spec=grid_spec,
        out_shape=jax.ShapeDtypeStruct((_M, _D), jnp.float32),
        input_output_aliases={1: 0},
    )(index, table, update)


def kernel(x, update, index):
    return _tc_fixup(x, update, index)


# speedup vs baseline: 21.6562x; 11.2241x over previous
import jax
import jax.numpy as jnp
from jax import lax
from jax.experimental import pallas as pl
from jax.experimental.pallas import tpu as pltpu
from jax.experimental.pallas import tpu_sc as plsc

_M = 1_000_000
_D = 8
_NW = 32
_CW = 3_968
_NCHUNK = 252
_TAIL0 = _NCHUNK * _CW
_TAILW = _M - _TAIL0
_BW = 128
_NSLOT = 8
_NBUF = 4
_LAG = _NBUF // 2
_NUPD = 4


def _copy_body(x_hbm, out_hbm, b0, b1, b2, b3, rsems, wsems):
    wid = lax.axis_index("s") * 2 + lax.axis_index("c")
    bufs = (b0, b1, b2, b3)

    def chunk_id(k):
        return wid + k * _NW

    def valid(k):
        return chunk_id(k) < _NCHUNK

    def rd(k):
        b = k % _NBUF
        col0 = chunk_id(k) * _CW
        return pltpu.make_async_copy(
            x_hbm.at[:, pl.ds(col0, _CW)], bufs[b], rsems.at[b])

    def wr(k):
        b = k % _NBUF
        col0 = chunk_id(k) * _CW
        return pltpu.make_async_copy(
            bufs[b], out_hbm.at[:, pl.ds(col0, _CW)], wsems.at[b])

    for k in range(_LAG):
        pl.when(valid(k))(lambda k=k: rd(k).start())
    for k in range(_NSLOT):
        if k >= _LAG:
            pl.when(valid(k - _LAG))(lambda k=k: wr(k - _LAG).wait())
        if k + _LAG < _NSLOT:
            pl.when(valid(k + _LAG))(lambda k=k: rd(k + _LAG).start())

        def _proc(k=k):
            rd(k).wait()
            wr(k).start()

        pl.when(valid(k))(_proc)
    for k in range(_NSLOT - _LAG, _NSLOT):
        pl.when(valid(k))(lambda k=k: wr(k).wait())


def _sc_copy(xt):
    mesh = plsc.VectorSubcoreMesh(
        core_axis_name="c", subcore_axis_name="s", num_cores=2, num_subcores=16
    )
    return pl.kernel(
        _copy_body,
        out_type=jax.ShapeDtypeStruct((_D, _M), jnp.float32),
        mesh=mesh,
        scratch_types=[
            pltpu.VMEM((_D, _CW), jnp.float32),
            pltpu.VMEM((_D, _CW), jnp.float32),
            pltpu.VMEM((_D, _CW), jnp.float32),
            pltpu.VMEM((_D, _CW), jnp.float32),
            pltpu.SemaphoreType.DMA((_NBUF,)),
            pltpu.SemaphoreType.DMA((_NBUF,)),
        ],
    )(xt)


def _fix_body(idx_ref, x_blk, updt_ref, out_blk):
    i = pl.program_id(0)
    bid = jnp.where(i == 0, _M // _BW,
                    idx_ref[jnp.maximum(i - 1, 0)] // _BW)
    col = bid * _BW + lax.broadcasted_iota(jnp.int32, (_D, _BW), 1)
    acc = x_blk[...]
    for j in range(_NUPD):
        acc = acc + jnp.where(col == idx_ref[j], updt_ref[:, pl.ds(j, 1)], 0.0)
    out_blk[...] = acc


def _block_map(i, idx_ref):
    return (0, jnp.where(i == 0, _M // _BW,
                         idx_ref[jnp.maximum(i - 1, 0)] // _BW))


def _tc_fixup(tabt, updt, xt, index):
    grid_spec = pltpu.PrefetchScalarGridSpec(
        num_scalar_prefetch=1,
        grid=(_NUPD + 1,),
        in_specs=[
            pl.BlockSpec((_D, _BW), _block_map),
            pl.BlockSpec((_D, _NUPD), lambda i, idx: (0, 0)),
        ],
        out_specs=pl.BlockSpec((_D, _BW), _block_map),
    )

    def body(idx_ref, x_blk, updt_ref, tab_ref, out_blk):
        del tab_ref
        _fix_body(idx_ref, x_blk, updt_ref, out_blk)

    grid_spec2 = pltpu.PrefetchScalarGridSpec(
        num_scalar_prefetch=1,
        grid=(_NUPD + 1,),
        in_specs=[
            pl.BlockSpec((_D, _BW), _block_map),
            pl.BlockSpec((_D, _NUPD), lambda i, idx: (0, 0)),
            pl.BlockSpec(memory_space=pl.ANY),
        ],
        out_specs=pl.BlockSpec((_D, _BW), _block_map),
    )
    return pl.pallas_call(
        body,
        grid_spec=grid_spec2,
        out_shape=jax.ShapeDtypeStruct((_D, _M), jnp.float32),
        input_output_aliases={3: 0},
    )(index, xt, updt, tabt)


def kernel(x, update, index):
    xt = x.T
    fixed = _tc_fixup(_sc_copy(xt), update.T, xt, index)
    return fixed.T
